# baseline (speedup 1.0000x reference)
import jax
import jax.numpy as jnp
from jax.experimental import pallas as pl
from jax.experimental.pallas import tpu as pltpu

_LANE = 128
_ROT_A = (13, 15, 26, 6)
_ROT_B = (17, 29, 16, 24)
_THREEFRY_C = 0x1BD11BDA
_KEEP_THRESHOLD = int(round(0.1 * 2.0 ** 32))
_KEEP_SCALE = 1.0 / (1.0 - 0.1)


def _round_up(x, m):
    return (x + m - 1) // m * m


def _rotl(x, r):
    return (x << jnp.uint32(r)) | (x >> jnp.uint32(32 - r))


def _threefry2x32(k0, k1, x0, x1):
    ks2 = k0 ^ k1 ^ jnp.uint32(_THREEFRY_C)
    ks = (k0, k1, ks2)
    x0 = x0 + k0
    x1 = x1 + k1
    for i in range(5):
        for r in (_ROT_A if i % 2 == 0 else _ROT_B):
            x0 = x0 + x1
            x1 = _rotl(x1, r)
            x1 = x0 ^ x1
        x0 = x0 + ks[(i + 1) % 3]
        x1 = x1 + (ks[(i + 2) % 3] + jnp.uint32(i + 1))
    return x0, x1


def _random_bits(k0, k1, idx):
    x0 = jnp.broadcast_to(k0, idx.shape)
    x1 = idx + k1
    ks2 = k0 ^ k1 ^ jnp.uint32(_THREEFRY_C)
    ks = (k0, k1, ks2)
    for i in range(5):
        for r in (_ROT_A if i % 2 == 0 else _ROT_B):
            x0 = x0 + x1
            x1 = _rotl(x1, r)
            x1 = x0 ^ x1
        x0 = x0 + ks[(i + 1) % 3]
        x1 = x1 + (ks[(i + 2) % 3] + jnp.uint32(i + 1))
    return x0 ^ x1


def _qa_kernel(keys_ref, x_ref, w_ref, bias_ref, o_ref, *, tm, hp, nl):
    i = pl.program_id(0)
    s1a = keys_ref[0]
    s1b = keys_ref[1]
    s2a = keys_ref[2]
    s2b = keys_ref[3]

    row = jax.lax.broadcasted_iota(jnp.int32, (x_ref.shape[0], hp), 0)
    col = jax.lax.broadcasted_iota(jnp.int32, (x_ref.shape[0], hp), 1)
    idx = ((i * tm + row) * hp + col).astype(jnp.uint32)
    thr = jnp.uint32(_KEEP_THRESHOLD)
    xs = x_ref[...] * _KEEP_SCALE
    x1 = jnp.where(_random_bits(s1a, s1b, idx) >= thr, xs, 0.0)
    x2 = jnp.where(_random_bits(s2a, s2b, idx) >= thr, xs, 0.0)

    o = jnp.dot(x1, w_ref[:, :_LANE], preferred_element_type=jnp.float32)
    o += jnp.dot(x2, w_ref[:, _LANE:], preferred_element_type=jnp.float32)
    o_ref[...] = o + bias_ref[...]


def kernel(hidden_states, w1, b1, w2, b2, dropout_key):
    B, S, H = hidden_states.shape
    nl = w1.shape[1]
    M = B * S
    x = hidden_states.reshape(M, H).astype(jnp.float32)

    TM = min(1536, _round_up(M, 8))
    Mp = _round_up(M, TM)
    Hp = _round_up(H, _LANE)
    if (Mp, Hp) != (M, H):
        x = jnp.zeros((Mp, Hp), jnp.float32).at[:M, :H].set(x)

    w = jnp.zeros((Hp, 2 * _LANE), jnp.float32)
    w = w.at[:H, :nl].set(w1.astype(jnp.float32))
    w = w.at[:H, _LANE + nl:_LANE + 2 * nl].set(w2.astype(jnp.float32))
    bias = jnp.zeros((1, _LANE), jnp.float32)
    bias = bias.at[0, :nl].set(b1.astype(jnp.float32))
    bias = bias.at[0, nl:2 * nl].set(b2.astype(jnp.float32))

    key = jax.random.wrap_key_data(dropout_key)
    k1, k2 = jax.random.split(key)
    keys4 = jnp.concatenate(
        [jax.random.key_data(k1), jax.random.key_data(k2)]).astype(jnp.uint32)

    grid = (Mp // TM,)
    o = pl.pallas_call(
        lambda *a: _qa_kernel(*a, tm=TM, hp=Hp, nl=nl),
        out_shape=jax.ShapeDtypeStruct((Mp, _LANE), jnp.float32),
        grid=grid,
        in_specs=[
            pl.BlockSpec(memory_space=pltpu.SMEM),
            pl.BlockSpec((TM, Hp), lambda i: (i, 0)),
            pl.BlockSpec((Hp, 2 * _LANE), lambda i: (0, 0)),
            pl.BlockSpec((1, _LANE), lambda i: (0, 0)),
        ],
        out_specs=pl.BlockSpec((TM, _LANE), lambda i: (i, 0)),
        compiler_params=pltpu.CompilerParams(
            dimension_semantics=("parallel",),
            vmem_limit_bytes=48 * 1024 * 1024,
        ),
    )(keys4, x, w, bias)

    start_logits = o[:M, :nl].reshape(B, S, nl)
    end_logits = o[:M, nl:2 * nl].reshape(B, S, nl)
    return start_logits, end_logits


# speedup vs baseline: 1.2920x; 1.2920x over previous
import jax
import jax.numpy as jnp
from jax.experimental import pallas as pl
from jax.experimental.pallas import tpu as pltpu

_LANE = 128
_ROT_A = (13, 15, 26, 6)
_ROT_B = (17, 29, 16, 24)
_THREEFRY_C = 0x1BD11BDA
_KEEP_THRESHOLD = int(round(0.1 * 2.0 ** 32))
_KEEP_SCALE = 1.0 / (1.0 - 0.1)


def _round_up(x, m):
    return (x + m - 1) // m * m


def _rotl(x, r):
    return (x << jnp.uint32(r)) | (x >> jnp.uint32(32 - r))


def _threefry2x32(k0, k1, x0, x1):
    ks2 = k0 ^ k1 ^ jnp.uint32(_THREEFRY_C)
    ks = (k0, k1, ks2)
    x0 = x0 + k0
    x1 = x1 + k1
    for i in range(5):
        for r in (_ROT_A if i % 2 == 0 else _ROT_B):
            x0 = x0 + x1
            x1 = _rotl(x1, r)
            x1 = x0 ^ x1
        x0 = x0 + ks[(i + 1) % 3]
        x1 = x1 + (ks[(i + 2) % 3] + jnp.uint32(i + 1))
    return x0, x1


def _random_bits(k0, k1, idx):
    x0 = jnp.broadcast_to(k0, idx.shape)
    x1 = idx + k1
    ks2 = k0 ^ k1 ^ jnp.uint32(_THREEFRY_C)
    ks = (k0, k1, ks2)
    for i in range(5):
        for r in (_ROT_A if i % 2 == 0 else _ROT_B):
            x0 = x0 + x1
            x1 = _rotl(x1, r)
            x1 = x0 ^ x1
        x0 = x0 + ks[(i + 1) % 3]
        x1 = x1 + (ks[(i + 2) % 3] + jnp.uint32(i + 1))
    return x0 ^ x1


def _qa_kernel(keys_ref, x_ref, w_ref, bias_ref, o_ref, *, tm, hp, nl):
    i = pl.program_id(0)
    s1a = keys_ref[0]
    s1b = keys_ref[1]
    s2a = keys_ref[2]
    s2b = keys_ref[3]

    row = jax.lax.broadcasted_iota(jnp.int32, (x_ref.shape[0], hp), 0)
    col = jax.lax.broadcasted_iota(jnp.int32, (x_ref.shape[0], hp), 1)
    idx = ((i * tm + row) * hp + col).astype(jnp.uint32)
    thr = jnp.uint32(_KEEP_THRESHOLD)
    xs = x_ref[...] * _KEEP_SCALE
    x1 = jnp.where(_random_bits(s1a, s1b, idx) >= thr, xs, 0.0)
    x2 = jnp.where(_random_bits(s2a, s2b, idx) >= thr, xs, 0.0)

    o = jnp.dot(x1, w_ref[:, :_LANE], preferred_element_type=jnp.float32)
    o += jnp.dot(x2, w_ref[:, _LANE:], preferred_element_type=jnp.float32)
    o_ref[...] = o + bias_ref[...]


def kernel(hidden_states, w1, b1, w2, b2, dropout_key):
    B, S, H = hidden_states.shape
    nl = w1.shape[1]
    M = B * S
    x = hidden_states.reshape(M, H).astype(jnp.float32)

    TM = min(512, _round_up(M, 8))
    Mp = _round_up(M, TM)
    Hp = _round_up(H, _LANE)
    if (Mp, Hp) != (M, H):
        x = jnp.zeros((Mp, Hp), jnp.float32).at[:M, :H].set(x)

    w = jnp.zeros((Hp, 2 * _LANE), jnp.float32)
    w = w.at[:H, :nl].set(w1.astype(jnp.float32))
    w = w.at[:H, _LANE + nl:_LANE + 2 * nl].set(w2.astype(jnp.float32))
    bias = jnp.zeros((1, _LANE), jnp.float32)
    bias = bias.at[0, :nl].set(b1.astype(jnp.float32))
    bias = bias.at[0, nl:2 * nl].set(b2.astype(jnp.float32))

    key = jax.random.wrap_key_data(dropout_key)
    k1, k2 = jax.random.split(key)
    keys4 = jnp.concatenate(
        [jax.random.key_data(k1), jax.random.key_data(k2)]).astype(jnp.uint32)

    grid = (Mp // TM,)
    o = pl.pallas_call(
        lambda *a: _qa_kernel(*a, tm=TM, hp=Hp, nl=nl),
        out_shape=jax.ShapeDtypeStruct((Mp, _LANE), jnp.float32),
        grid=grid,
        in_specs=[
            pl.BlockSpec(memory_space=pltpu.SMEM),
            pl.BlockSpec((TM, Hp), lambda i: (i, 0)),
            pl.BlockSpec((Hp, 2 * _LANE), lambda i: (0, 0)),
            pl.BlockSpec((1, _LANE), lambda i: (0, 0)),
        ],
        out_specs=pl.BlockSpec((TM, _LANE), lambda i: (i, 0)),
        compiler_params=pltpu.CompilerParams(
            dimension_semantics=("parallel",),
            vmem_limit_bytes=48 * 1024 * 1024,
        ),
    )(keys4, x, w, bias)

    start_logits = o[:M, :nl].reshape(B, S, nl)
    end_logits = o[:M, nl:2 * nl].reshape(B, S, nl)
    return start_logits, end_logits


# speedup vs baseline: 1.3002x; 1.0064x over previous
import jax
import jax.numpy as jnp
from jax.experimental import pallas as pl
from jax.experimental.pallas import tpu as pltpu

_LANE = 128
_ROT_A = (13, 15, 26, 6)
_ROT_B = (17, 29, 16, 24)
_THREEFRY_C = 0x1BD11BDA
_KEEP_THRESHOLD = int(round(0.1 * 2.0 ** 32))
_KEEP_SCALE = 1.0 / (1.0 - 0.1)


def _round_up(x, m):
    return (x + m - 1) // m * m


def _rotl(x, r):
    return (x << jnp.uint32(r)) | (x >> jnp.uint32(32 - r))


def _threefry2x32(k0, k1, x0, x1):
    ks2 = k0 ^ k1 ^ jnp.uint32(_THREEFRY_C)
    ks = (k0, k1, ks2)
    x0 = x0 + k0
    x1 = x1 + k1
    for i in range(5):
        for r in (_ROT_A if i % 2 == 0 else _ROT_B):
            x0 = x0 + x1
            x1 = _rotl(x1, r)
            x1 = x0 ^ x1
        x0 = x0 + ks[(i + 1) % 3]
        x1 = x1 + (ks[(i + 2) % 3] + jnp.uint32(i + 1))
    return x0, x1


def _random_bits(k0, k1, idx):
    x0 = jnp.broadcast_to(k0, idx.shape)
    x1 = idx + k1
    ks2 = k0 ^ k1 ^ jnp.uint32(_THREEFRY_C)
    ks = (k0, k1, ks2)
    for i in range(5):
        for r in (_ROT_A if i % 2 == 0 else _ROT_B):
            x0 = x0 + x1
            x1 = _rotl(x1, r)
            x1 = x0 ^ x1
        x0 = x0 + ks[(i + 1) % 3]
        x1 = x1 + (ks[(i + 2) % 3] + jnp.uint32(i + 1))
    return x0 ^ x1


def _qa_kernel(keys_ref, x_ref, w_ref, bias_ref, o_ref, *, tm, hp, nl):
    i = pl.program_id(0)
    k0 = keys_ref[0]
    k1 = keys_ref[1]
    z = jnp.uint32(0)
    s1a, s1b = _threefry2x32(k0, k1, z, z)
    s2a, s2b = _threefry2x32(k0, k1, z, jnp.uint32(1))

    row = jax.lax.broadcasted_iota(jnp.int32, (x_ref.shape[0], hp), 0)
    col = jax.lax.broadcasted_iota(jnp.int32, (x_ref.shape[0], hp), 1)
    idx = ((i * tm + row) * hp + col).astype(jnp.uint32)
    thr = jnp.uint32(_KEEP_THRESHOLD)
    xs = x_ref[...] * _KEEP_SCALE
    x1 = jnp.where(_random_bits(s1a, s1b, idx) >= thr, xs, 0.0)
    x2 = jnp.where(_random_bits(s2a, s2b, idx) >= thr, xs, 0.0)

    o = jnp.dot(x1, w_ref[:, :_LANE], preferred_element_type=jnp.float32)
    o += jnp.dot(x2, w_ref[:, _LANE:], preferred_element_type=jnp.float32)
    o_ref[...] = o + bias_ref[...]


def kernel(hidden_states, w1, b1, w2, b2, dropout_key):
    B, S, H = hidden_states.shape
    nl = w1.shape[1]
    M = B * S
    x = hidden_states.reshape(M, H).astype(jnp.float32)

    TM = min(512, _round_up(M, 8))
    Mp = _round_up(M, TM)
    Hp = _round_up(H, _LANE)
    if (Mp, Hp) != (M, H):
        x = jnp.zeros((Mp, Hp), jnp.float32).at[:M, :H].set(x)

    w = jnp.zeros((Hp, 2 * _LANE), jnp.float32)
    w = w.at[:H, :nl].set(w1.astype(jnp.float32))
    w = w.at[:H, _LANE + nl:_LANE + 2 * nl].set(w2.astype(jnp.float32))
    bias = jnp.zeros((1, _LANE), jnp.float32)
    bias = bias.at[0, :nl].set(b1.astype(jnp.float32))
    bias = bias.at[0, nl:2 * nl].set(b2.astype(jnp.float32))

    grid = (Mp // TM,)
    o = pl.pallas_call(
        lambda *a: _qa_kernel(*a, tm=TM, hp=Hp, nl=nl),
        out_shape=jax.ShapeDtypeStruct((Mp, _LANE), jnp.float32),
        grid=grid,
        in_specs=[
            pl.BlockSpec(memory_space=pltpu.SMEM),
            pl.BlockSpec((TM, Hp), lambda i: (i, 0)),
            pl.BlockSpec((Hp, 2 * _LANE), lambda i: (0, 0)),
            pl.BlockSpec((1, _LANE), lambda i: (0, 0)),
        ],
        out_specs=pl.BlockSpec((TM, _LANE), lambda i: (i, 0)),
        compiler_params=pltpu.CompilerParams(
            dimension_semantics=("parallel",),
            vmem_limit_bytes=48 * 1024 * 1024,
        ),
    )(dropout_key.astype(jnp.uint32), x, w, bias)

    start_logits = o[:M, :nl].reshape(B, S, nl)
    end_logits = o[:M, nl:2 * nl].reshape(B, S, nl)
    return start_logits, end_logits


# speedup vs baseline: 1.3029x; 1.0021x over previous
import jax
import jax.numpy as jnp
from jax.experimental import pallas as pl
from jax.experimental.pallas import tpu as pltpu

_LANE = 128
_ROT_A = (13, 15, 26, 6)
_ROT_B = (17, 29, 16, 24)
_THREEFRY_C = 0x1BD11BDA
_KEEP_THRESHOLD = int(round(0.1 * 2.0 ** 32))
_KEEP_SCALE = 1.0 / (1.0 - 0.1)


def _round_up(x, m):
    return (x + m - 1) // m * m


def _rotl(x, r):
    return (x << jnp.uint32(r)) | (x >> jnp.uint32(32 - r))


def _threefry2x32(k0, k1, x0, x1):
    ks2 = k0 ^ k1 ^ jnp.uint32(_THREEFRY_C)
    ks = (k0, k1, ks2)
    x0 = x0 + k0
    x1 = x1 + k1
    for i in range(5):
        for r in (_ROT_A if i % 2 == 0 else _ROT_B):
            x0 = x0 + x1
            x1 = _rotl(x1, r)
            x1 = x0 ^ x1
        x0 = x0 + ks[(i + 1) % 3]
        x1 = x1 + (ks[(i + 2) % 3] + jnp.uint32(i + 1))
    return x0, x1


def _random_bits(k0, k1, idx):
    x0 = jnp.broadcast_to(k0, idx.shape)
    x1 = idx + k1
    ks2 = k0 ^ k1 ^ jnp.uint32(_THREEFRY_C)
    ks = (k0, k1, ks2)
    for i in range(5):
        for r in (_ROT_A if i % 2 == 0 else _ROT_B):
            x0 = x0 + x1
            x1 = _rotl(x1, r)
            x1 = x0 ^ x1
        x0 = x0 + ks[(i + 1) % 3]
        x1 = x1 + (ks[(i + 2) % 3] + jnp.uint32(i + 1))
    return x0 ^ x1


def _qa_kernel(keys_ref, x_ref, w_ref, bias_ref, o_ref, *, tm, hp, nl):
    i = pl.program_id(0)
    k0 = keys_ref[0]
    k1 = keys_ref[1]
    z = jnp.uint32(0)
    s1a, s1b = _threefry2x32(k0, k1, z, z)
    s2a, s2b = _threefry2x32(k0, k1, z, jnp.uint32(1))

    row = jax.lax.broadcasted_iota(jnp.int32, (x_ref.shape[0], hp), 0)
    col = jax.lax.broadcasted_iota(jnp.int32, (x_ref.shape[0], hp), 1)
    idx = ((i * tm + row) * hp + col).astype(jnp.uint32)
    thr = jnp.uint32(_KEEP_THRESHOLD)
    xs = x_ref[...] * _KEEP_SCALE
    x1 = jnp.where(_random_bits(s1a, s1b, idx) >= thr, xs, 0.0)
    x2 = jnp.where(_random_bits(s2a, s2b, idx) >= thr, xs, 0.0)

    o = jnp.dot(x1, w_ref[:, :_LANE], preferred_element_type=jnp.float32)
    o += jnp.dot(x2, w_ref[:, _LANE:], preferred_element_type=jnp.float32)
    o_ref[...] = o + bias_ref[...]


def kernel(hidden_states, w1, b1, w2, b2, dropout_key):
    B, S, H = hidden_states.shape
    nl = w1.shape[1]
    M = B * S
    x = hidden_states.reshape(M, H).astype(jnp.float32)

    TM = min(768, _round_up(M, 8))
    Mp = _round_up(M, TM)
    Hp = _round_up(H, _LANE)
    if (Mp, Hp) != (M, H):
        x = jnp.zeros((Mp, Hp), jnp.float32).at[:M, :H].set(x)

    w = jnp.zeros((Hp, 2 * _LANE), jnp.float32)
    w = w.at[:H, :nl].set(w1.astype(jnp.float32))
    w = w.at[:H, _LANE + nl:_LANE + 2 * nl].set(w2.astype(jnp.float32))
    bias = jnp.zeros((1, _LANE), jnp.float32)
    bias = bias.at[0, :nl].set(b1.astype(jnp.float32))
    bias = bias.at[0, nl:2 * nl].set(b2.astype(jnp.float32))

    grid = (Mp // TM,)
    o = pl.pallas_call(
        lambda *a: _qa_kernel(*a, tm=TM, hp=Hp, nl=nl),
        out_shape=jax.ShapeDtypeStruct((Mp, _LANE), jnp.float32),
        grid=grid,
        in_specs=[
            pl.BlockSpec(memory_space=pltpu.SMEM),
            pl.BlockSpec((TM, Hp), lambda i: (i, 0)),
            pl.BlockSpec((Hp, 2 * _LANE), lambda i: (0, 0)),
            pl.BlockSpec((1, _LANE), lambda i: (0, 0)),
        ],
        out_specs=pl.BlockSpec((TM, _LANE), lambda i: (i, 0)),
        compiler_params=pltpu.CompilerParams(
            dimension_semantics=("parallel",),
            vmem_limit_bytes=48 * 1024 * 1024,
        ),
    )(dropout_key.astype(jnp.uint32), x, w, bias)

    start_logits = o[:M, :nl].reshape(B, S, nl)
    end_logits = o[:M, nl:2 * nl].reshape(B, S, nl)
    return start_logits, end_logits


# speedup vs baseline: 1.3522x; 1.0378x over previous
import jax
import jax.numpy as jnp
from jax.experimental import pallas as pl
from jax.experimental.pallas import tpu as pltpu

_LANE = 128
_ROT_A = (13, 15, 26, 6)
_ROT_B = (17, 29, 16, 24)
_THREEFRY_C = 0x1BD11BDA
_KEEP_THRESHOLD = int(round(0.1 * 2.0 ** 32))
_KEEP_SCALE = 1.0 / (1.0 - 0.1)


def _round_up(x, m):
    return (x + m - 1) // m * m


def _rotl(x, r):
    return (x << jnp.uint32(r)) | (x >> jnp.uint32(32 - r))


def _threefry2x32(k0, k1, x0, x1):
    ks2 = k0 ^ k1 ^ jnp.uint32(_THREEFRY_C)
    ks = (k0, k1, ks2)
    x0 = x0 + k0
    x1 = x1 + k1
    for i in range(5):
        for r in (_ROT_A if i % 2 == 0 else _ROT_B):
            x0 = x0 + x1
            x1 = _rotl(x1, r)
            x1 = x0 ^ x1
        x0 = x0 + ks[(i + 1) % 3]
        x1 = x1 + (ks[(i + 2) % 3] + jnp.uint32(i + 1))
    return x0, x1


def _random_bits(k0, k1, idx):
    x0 = jnp.broadcast_to(k0, idx.shape)
    x1 = idx + k1
    ks2 = k0 ^ k1 ^ jnp.uint32(_THREEFRY_C)
    ks = (k0, k1, ks2)
    for i in range(5):
        for r in (_ROT_A if i % 2 == 0 else _ROT_B):
            x0 = x0 + x1
            x1 = _rotl(x1, r)
            x1 = x0 ^ x1
        x0 = x0 + ks[(i + 1) % 3]
        x1 = x1 + (ks[(i + 2) % 3] + jnp.uint32(i + 1))
    return x0 ^ x1


def _qa_kernel(keys_ref, x_ref, w1_ref, w2_ref, b_ref, o_ref, *, tm, hp, nl):
    i = pl.program_id(0)
    k0 = keys_ref[0]
    k1 = keys_ref[1]
    z = jnp.uint32(0)
    s1a, s1b = _threefry2x32(k0, k1, z, z)
    s2a, s2b = _threefry2x32(k0, k1, z, jnp.uint32(1))

    row = jax.lax.broadcasted_iota(jnp.int32, (x_ref.shape[0], hp), 0)
    col = jax.lax.broadcasted_iota(jnp.int32, (x_ref.shape[0], hp), 1)
    idx = ((i * tm + row) * hp + col).astype(jnp.uint32)
    thr = jnp.uint32(_KEEP_THRESHOLD)
    xs = x_ref[...] * _KEEP_SCALE
    x1 = jnp.where(_random_bits(s1a, s1b, idx) >= thr, xs, 0.0)
    x2 = jnp.where(_random_bits(s2a, s2b, idx) >= thr, xs, 0.0)

    zpad = jnp.zeros((hp, _LANE - 2 * nl), jnp.float32)
    w1p = jnp.concatenate(
        [w1_ref[...], jnp.zeros((hp, nl), jnp.float32), zpad], axis=1)
    w2p = jnp.concatenate(
        [jnp.zeros((hp, nl), jnp.float32), w2_ref[...], zpad], axis=1)

    lane = jax.lax.broadcasted_iota(jnp.int32, (1, _LANE), 1)
    bias = jnp.zeros((1, _LANE), jnp.float32)
    for j in range(2 * nl):
        bias = jnp.where(lane == j, b_ref[j], bias)

    o = jnp.dot(x1, w1p, preferred_element_type=jnp.float32)
    o += jnp.dot(x2, w2p, preferred_element_type=jnp.float32)
    o_ref[...] = o + bias


def kernel(hidden_states, w1, b1, w2, b2, dropout_key):
    B, S, H = hidden_states.shape
    nl = w1.shape[1]
    M = B * S
    x = hidden_states.reshape(M, H).astype(jnp.float32)

    TM = min(768, _round_up(M, 8))
    Mp = _round_up(M, TM)
    Hp = _round_up(H, _LANE)
    if (Mp, Hp) != (M, H):
        x = jnp.zeros((Mp, Hp), jnp.float32).at[:M, :H].set(x)

    w1f = w1.astype(jnp.float32)
    w2f = w2.astype(jnp.float32)
    if Hp != H:
        w1f = jnp.zeros((Hp, nl), jnp.float32).at[:H].set(w1f)
        w2f = jnp.zeros((Hp, nl), jnp.float32).at[:H].set(w2f)
    bcat = jnp.concatenate([b1.astype(jnp.float32), b2.astype(jnp.float32)])

    grid = (Mp // TM,)
    o = pl.pallas_call(
        lambda *a: _qa_kernel(*a, tm=TM, hp=Hp, nl=nl),
        out_shape=jax.ShapeDtypeStruct((Mp, _LANE), jnp.float32),
        grid=grid,
        in_specs=[
            pl.BlockSpec(memory_space=pltpu.SMEM),
            pl.BlockSpec((TM, Hp), lambda i: (i, 0)),
            pl.BlockSpec((Hp, nl), lambda i: (0, 0)),
            pl.BlockSpec((Hp, nl), lambda i: (0, 0)),
            pl.BlockSpec(memory_space=pltpu.SMEM),
        ],
        out_specs=pl.BlockSpec((TM, _LANE), lambda i: (i, 0)),
        compiler_params=pltpu.CompilerParams(
            dimension_semantics=("parallel",),
            vmem_limit_bytes=48 * 1024 * 1024,
        ),
    )(dropout_key.astype(jnp.uint32), x, w1f, w2f, bcat)

    start_logits = o[:M, :nl].reshape(B, S, nl)
    end_logits = o[:M, nl:2 * nl].reshape(B, S, nl)
    return start_logits, end_logits


# speedup vs baseline: 1.3549x; 1.0020x over previous
import jax
import jax.numpy as jnp
from jax.experimental import pallas as pl
from jax.experimental.pallas import tpu as pltpu

_LANE = 128
_ROT_A = (13, 15, 26, 6)
_ROT_B = (17, 29, 16, 24)
_THREEFRY_C = 0x1BD11BDA
_KEEP_THRESHOLD = int(round(0.1 * 2.0 ** 32))
_KEEP_SCALE = 1.0 / (1.0 - 0.1)


def _round_up(x, m):
    return (x + m - 1) // m * m


def _rotl(x, r):
    return (x << jnp.uint32(r)) | (x >> jnp.uint32(32 - r))


def _threefry2x32(k0, k1, x0, x1):
    ks2 = k0 ^ k1 ^ jnp.uint32(_THREEFRY_C)
    ks = (k0, k1, ks2)
    x0 = x0 + k0
    x1 = x1 + k1
    for i in range(5):
        for r in (_ROT_A if i % 2 == 0 else _ROT_B):
            x0 = x0 + x1
            x1 = _rotl(x1, r)
            x1 = x0 ^ x1
        x0 = x0 + ks[(i + 1) % 3]
        x1 = x1 + (ks[(i + 2) % 3] + jnp.uint32(i + 1))
    return x0, x1


def _random_bits(k0, k1, x1):
    x0 = jnp.broadcast_to(k0, x1.shape)
    ks2 = k0 ^ k1 ^ jnp.uint32(_THREEFRY_C)
    ks = (k0, k1, ks2)
    for i in range(5):
        for r in (_ROT_A if i % 2 == 0 else _ROT_B):
            x0 = x0 + x1
            x1 = _rotl(x1, r)
            x1 = x0 ^ x1
        x0 = x0 + ks[(i + 1) % 3]
        x1 = x1 + (ks[(i + 2) % 3] + jnp.uint32(i + 1))
    return x0 ^ x1


def _qa_kernel(keys_ref, x_ref, w1_ref, w2_ref, b_ref, o_ref, *, tm, hp, nl):
    i = pl.program_id(0)
    k0 = keys_ref[0]
    k1 = keys_ref[1]
    z = jnp.uint32(0)
    s1a, s1b = _threefry2x32(k0, k1, z, z)
    s2a, s2b = _threefry2x32(k0, k1, z, jnp.uint32(1))

    row = jax.lax.broadcasted_iota(jnp.int32, (x_ref.shape[0], hp), 0)
    col = jax.lax.broadcasted_iota(jnp.int32, (x_ref.shape[0], hp), 1)
    pattern = (row * hp + col).astype(jnp.uint32)
    base = jnp.uint32(i * tm * hp)
    thr = jnp.uint32(_KEEP_THRESHOLD)
    xs = x_ref[...] * _KEEP_SCALE
    x1 = jnp.where(
        _random_bits(s1a, s1b, pattern + (s1b + base)) >= thr, xs, 0.0)
    x2 = jnp.where(
        _random_bits(s2a, s2b, pattern + (s2b + base)) >= thr, xs, 0.0)

    zpad = jnp.zeros((hp, _LANE - 2 * nl), jnp.float32)
    w1p = jnp.concatenate(
        [w1_ref[...], jnp.zeros((hp, nl), jnp.float32), zpad], axis=1)
    w2p = jnp.concatenate(
        [jnp.zeros((hp, nl), jnp.float32), w2_ref[...], zpad], axis=1)

    lane = jax.lax.broadcasted_iota(jnp.int32, (1, _LANE), 1)
    bias = jnp.zeros((1, _LANE), jnp.float32)
    for j in range(2 * nl):
        bias = jnp.where(lane == j, b_ref[j], bias)

    o = jnp.dot(x1, w1p, preferred_element_type=jnp.float32)
    o += jnp.dot(x2, w2p, preferred_element_type=jnp.float32)
    o_ref[...] = o + bias


def kernel(hidden_states, w1, b1, w2, b2, dropout_key):
    B, S, H = hidden_states.shape
    nl = w1.shape[1]
    M = B * S
    x = hidden_states.reshape(M, H).astype(jnp.float32)

    TM = min(1024, _round_up(M, 8))
    Mp = _round_up(M, TM)
    Hp = _round_up(H, _LANE)
    if (Mp, Hp) != (M, H):
        x = jnp.zeros((Mp, Hp), jnp.float32).at[:M, :H].set(x)

    w1f = w1.astype(jnp.float32)
    w2f = w2.astype(jnp.float32)
    if Hp != H:
        w1f = jnp.zeros((Hp, nl), jnp.float32).at[:H].set(w1f)
        w2f = jnp.zeros((Hp, nl), jnp.float32).at[:H].set(w2f)
    bcat = jnp.concatenate([b1.astype(jnp.float32), b2.astype(jnp.float32)])

    grid = (Mp // TM,)
    o = pl.pallas_call(
        lambda *a: _qa_kernel(*a, tm=TM, hp=Hp, nl=nl),
        out_shape=jax.ShapeDtypeStruct((Mp, _LANE), jnp.float32),
        grid=grid,
        in_specs=[
            pl.BlockSpec(memory_space=pltpu.SMEM),
            pl.BlockSpec((TM, Hp), lambda i: (i, 0)),
            pl.BlockSpec((Hp, nl), lambda i: (0, 0)),
            pl.BlockSpec((Hp, nl), lambda i: (0, 0)),
            pl.BlockSpec(memory_space=pltpu.SMEM),
        ],
        out_specs=pl.BlockSpec((TM, _LANE), lambda i: (i, 0)),
        compiler_params=pltpu.CompilerParams(
            dimension_semantics=("parallel",),
            vmem_limit_bytes=48 * 1024 * 1024,
        ),
    )(dropout_key.astype(jnp.uint32), x, w1f, w2f, bcat)

    start_logits = o[:M, :nl].reshape(B, S, nl)
    end_logits = o[:M, nl:2 * nl].reshape(B, S, nl)
    return start_logits, end_logits
